# baseline (speedup 1.0000x reference)
import functools

import jax
import jax.numpy as jnp
from jax import lax
from jax.experimental import pallas as pl
from jax.experimental.pallas import tpu as pltpu
from jax.experimental.pallas import tpu_sc as plsc

NB_LINKS = 1024
K = 1024
D = 64
NB_Z = 16
H = 2048
BATCH = 1024

MBLK = 128
VQBLK = 2048


def _mlp3_body(x_ref, W0_ref, b0_ref, W1_ref, b1_ref, W2_ref, b2_ref, out_ref):
    dn = (((1,), (1,)), ((), ()))
    h = lax.dot_general(x_ref[...], W0_ref[...], dn,
                        preferred_element_type=jnp.float32)
    h = jnp.maximum(h + b0_ref[...][None, :], 0.0)
    h = lax.dot_general(h, W1_ref[...], dn, preferred_element_type=jnp.float32)
    h = jnp.maximum(h + b1_ref[...][None, :], 0.0)
    h = lax.dot_general(h, W2_ref[...], dn, preferred_element_type=jnp.float32)
    out_ref[...] = h + b2_ref[...][None, :]


def _mlp3_call(x, W0, b0, W1, b1, W2, b2):
    m = x.shape[0]
    out_d = W2.shape[0]
    grid = (m // MBLK,)
    return pl.pallas_call(
        _mlp3_body,
        grid=grid,
        in_specs=[
            pl.BlockSpec((MBLK, x.shape[1]), lambda i: (i, 0)),
            pl.BlockSpec(W0.shape, lambda i: (0, 0)),
            pl.BlockSpec(b0.shape, lambda i: (0,)),
            pl.BlockSpec(W1.shape, lambda i: (0, 0)),
            pl.BlockSpec(b1.shape, lambda i: (0,)),
            pl.BlockSpec(W2.shape, lambda i: (0, 0)),
            pl.BlockSpec(b2.shape, lambda i: (0,)),
        ],
        out_specs=pl.BlockSpec((MBLK, out_d), lambda i: (i, 0)),
        out_shape=jax.ShapeDtypeStruct((m, out_d), jnp.float32),
        compiler_params=pltpu.CompilerParams(
            vmem_limit_bytes=110 * 1024 * 1024,
        ),
    )(x, W0, b0, W1, b1, W2, b2)


def _dec_body(ze_ref, zq_ref, W0_ref, b0_ref, W1_ref, b1_ref, W2_ref, b2_ref,
              st_ref, out_ref):
    ze = ze_ref[...]
    zq = zq_ref[...]
    st = ze + (zq - ze)
    st_ref[...] = st
    dn = (((1,), (1,)), ((), ()))
    h = lax.dot_general(st, W0_ref[...], dn, preferred_element_type=jnp.float32)
    h = jnp.maximum(h + b0_ref[...][None, :], 0.0)
    h = lax.dot_general(h, W1_ref[...], dn, preferred_element_type=jnp.float32)
    h = jnp.maximum(h + b1_ref[...][None, :], 0.0)
    h = lax.dot_general(h, W2_ref[...], dn, preferred_element_type=jnp.float32)
    out_ref[...] = h + b2_ref[...][None, :]


def _dec_call(ze, zq, W0, b0, W1, b1, W2, b2):
    m = ze.shape[0]
    out_d = W2.shape[0]
    grid = (m // MBLK,)
    return pl.pallas_call(
        _dec_body,
        grid=grid,
        in_specs=[
            pl.BlockSpec((MBLK, ze.shape[1]), lambda i: (i, 0)),
            pl.BlockSpec((MBLK, zq.shape[1]), lambda i: (i, 0)),
            pl.BlockSpec(W0.shape, lambda i: (0, 0)),
            pl.BlockSpec(b0.shape, lambda i: (0,)),
            pl.BlockSpec(W1.shape, lambda i: (0, 0)),
            pl.BlockSpec(b1.shape, lambda i: (0,)),
            pl.BlockSpec(W2.shape, lambda i: (0, 0)),
            pl.BlockSpec(b2.shape, lambda i: (0,)),
        ],
        out_specs=[
            pl.BlockSpec((MBLK, ze.shape[1]), lambda i: (i, 0)),
            pl.BlockSpec((MBLK, out_d), lambda i: (i, 0)),
        ],
        out_shape=[
            jax.ShapeDtypeStruct((m, ze.shape[1]), jnp.float32),
            jax.ShapeDtypeStruct((m, out_d), jnp.float32),
        ],
        compiler_params=pltpu.CompilerParams(
            vmem_limit_bytes=110 * 1024 * 1024,
        ),
    )(ze, zq, W0, b0, W1, b1, W2, b2)


def _rowsq64(x2):
    p = x2[:, 0:8]
    for j in range(1, 8):
        p = p + x2[:, 8 * j:8 * j + 8]
    t = p[:, 0:4] + p[:, 4:8]
    t = t[:, 0:2] + t[:, 2:4]
    return t[:, 0:1] + t[:, 1:2]


def _colsq64(y2):
    p = y2[0:8, :]
    for j in range(1, 8):
        p = p + y2[8 * j:8 * j + 8, :]
    t = p[0:4, :] + p[4:8, :]
    t = t[0:2, :] + t[2:4, :]
    return t[0:1, :] + t[1:2, :]


def _vq_body(f_ref, cb_ref, cbT_ref, idx_ref):
    f = f_ref[...]
    cb = cb_ref[...]
    cbT = cbT_ref[...]
    dn = (((1,), (1,)), ((), ()))
    mm = lax.dot_general(f, cb, dn, preferred_element_type=jnp.float32)
    ss = _rowsq64(f * f)
    cc = _colsq64(cbT * cbT)
    d2 = ss + cc - 2.0 * mm
    dist = jnp.sqrt(jnp.maximum(d2, 0.0))
    m = jnp.min(dist, axis=1, keepdims=True)
    iota = lax.broadcasted_iota(jnp.int32, dist.shape, 1)
    hit = jnp.where(dist == m, iota, jnp.int32(dist.shape[1]))
    idx_ref[...] = jnp.min(hit, axis=1).astype(jnp.int32)


def _vq_call(flat, cb, cbT):
    n = flat.shape[0]
    grid = (n // VQBLK,)
    return pl.pallas_call(
        _vq_body,
        grid=grid,
        in_specs=[
            pl.BlockSpec((VQBLK, D), lambda i: (i, 0)),
            pl.BlockSpec((K, D), lambda i: (0, 0)),
            pl.BlockSpec((D, K), lambda i: (0, 0)),
        ],
        out_specs=pl.BlockSpec((VQBLK,), lambda i: (i,)),
        out_shape=jax.ShapeDtypeStruct((n,), jnp.int32),
        compiler_params=pltpu.CompilerParams(
            vmem_limit_bytes=110 * 1024 * 1024,
        ),
    )(flat, cb, cbT)


def _sc_gather(cb, idx):
    info = plsc.get_sparse_core_info()
    nw = info.num_cores * info.num_subcores
    n = idx.shape[0]
    b_per_w = n // nw
    mesh = plsc.VectorSubcoreMesh(core_axis_name="c", subcore_axis_name="s")

    @functools.partial(
        pl.kernel,
        mesh=mesh,
        out_type=jax.ShapeDtypeStruct((n, D), jnp.float32),
        scratch_types=[
            pltpu.VMEM((b_per_w,), jnp.int32),
            pltpu.VMEM((b_per_w, D), jnp.float32),
            pltpu.SemaphoreType.DMA,
        ],
        compiler_params=pltpu.CompilerParams(use_tc_tiling_on_sc=False),
    )
    def gather_kernel(table_hbm, idx_hbm, out_hbm, idx_v, rows_v, sem):
        wid = lax.axis_index("s") * info.num_cores + lax.axis_index("c")
        base = wid * b_per_w
        pltpu.sync_copy(idx_hbm.at[pl.ds(base, b_per_w)], idx_v)
        pltpu.async_copy(table_hbm.at[idx_v], rows_v, sem).wait()
        pltpu.sync_copy(rows_v, out_hbm.at[pl.ds(base, b_per_w)])

    return gather_kernel(cb, idx)


def kernel(x, codebook, eW0, eb0, eW1, eb1, eW2, eb2,
           dW0, db0, dW1, db1, dW2, db2):
    ze_flat = _mlp3_call(x, eW0, eb0, eW1, eb1, eW2, eb2)
    flat = ze_flat.reshape(-1, D)
    indices = _vq_call(flat, codebook, codebook.T)
    zq_flat = _sc_gather(codebook, indices)
    st_flat, x_hat = _dec_call(ze_flat, zq_flat.reshape(BATCH, NB_Z * D),
                               dW0, db0, dW1, db1, dW2, db2)
    shape3 = (BATCH, NB_Z, D)
    return (ze_flat.reshape(shape3),
            st_flat.reshape(shape3),
            zq_flat.reshape(shape3),
            x_hat,
            indices.reshape(BATCH, NB_Z))


# speedup vs baseline: 1.4896x; 1.4896x over previous
import functools

import jax
import jax.numpy as jnp
from jax import lax
from jax.experimental import pallas as pl
from jax.experimental.pallas import tpu as pltpu
from jax.experimental.pallas import tpu_sc as plsc

NB_LINKS = 1024
K = 1024
D = 64
NB_Z = 16
H = 2048
BATCH = 1024

MBLK = 256
VQBLK = 2048


def _mlp3_body(x_ref, W0_ref, b0_ref, W1_ref, b1_ref, W2_ref, b2_ref, out_ref):
    dn = (((1,), (1,)), ((), ()))
    h = lax.dot_general(x_ref[...], W0_ref[...], dn,
                        preferred_element_type=jnp.float32)
    h = jnp.maximum(h + b0_ref[...][None, :], 0.0)
    h = lax.dot_general(h, W1_ref[...], dn, preferred_element_type=jnp.float32)
    h = jnp.maximum(h + b1_ref[...][None, :], 0.0)
    h = lax.dot_general(h, W2_ref[...], dn, preferred_element_type=jnp.float32)
    out_ref[...] = h + b2_ref[...][None, :]


def _mlp3_call(x, W0, b0, W1, b1, W2, b2):
    m = x.shape[0]
    out_d = W2.shape[0]
    grid = (m // MBLK,)
    return pl.pallas_call(
        _mlp3_body,
        grid=grid,
        in_specs=[
            pl.BlockSpec((MBLK, x.shape[1]), lambda i: (i, 0)),
            pl.BlockSpec(W0.shape, lambda i: (0, 0)),
            pl.BlockSpec(b0.shape, lambda i: (0,)),
            pl.BlockSpec(W1.shape, lambda i: (0, 0)),
            pl.BlockSpec(b1.shape, lambda i: (0,)),
            pl.BlockSpec(W2.shape, lambda i: (0, 0)),
            pl.BlockSpec(b2.shape, lambda i: (0,)),
        ],
        out_specs=pl.BlockSpec((MBLK, out_d), lambda i: (i, 0)),
        out_shape=jax.ShapeDtypeStruct((m, out_d), jnp.float32),
        compiler_params=pltpu.CompilerParams(
            vmem_limit_bytes=110 * 1024 * 1024,
        ),
    )(x, W0, b0, W1, b1, W2, b2)


def _dec_body(ze_ref, zq_ref, W0_ref, b0_ref, W1_ref, b1_ref, W2_ref, b2_ref,
              st_ref, out_ref):
    ze = ze_ref[...]
    zq = zq_ref[...]
    st = ze + (zq - ze)
    st_ref[...] = st
    dn = (((1,), (1,)), ((), ()))
    h = lax.dot_general(st, W0_ref[...], dn, preferred_element_type=jnp.float32)
    h = jnp.maximum(h + b0_ref[...][None, :], 0.0)
    h = lax.dot_general(h, W1_ref[...], dn, preferred_element_type=jnp.float32)
    h = jnp.maximum(h + b1_ref[...][None, :], 0.0)
    h = lax.dot_general(h, W2_ref[...], dn, preferred_element_type=jnp.float32)
    out_ref[...] = h + b2_ref[...][None, :]


def _dec_call(ze, zq, W0, b0, W1, b1, W2, b2):
    m = ze.shape[0]
    out_d = W2.shape[0]
    grid = (m // MBLK,)
    return pl.pallas_call(
        _dec_body,
        grid=grid,
        in_specs=[
            pl.BlockSpec((MBLK, ze.shape[1]), lambda i: (i, 0)),
            pl.BlockSpec((MBLK, zq.shape[1]), lambda i: (i, 0)),
            pl.BlockSpec(W0.shape, lambda i: (0, 0)),
            pl.BlockSpec(b0.shape, lambda i: (0,)),
            pl.BlockSpec(W1.shape, lambda i: (0, 0)),
            pl.BlockSpec(b1.shape, lambda i: (0,)),
            pl.BlockSpec(W2.shape, lambda i: (0, 0)),
            pl.BlockSpec(b2.shape, lambda i: (0,)),
        ],
        out_specs=[
            pl.BlockSpec((MBLK, ze.shape[1]), lambda i: (i, 0)),
            pl.BlockSpec((MBLK, out_d), lambda i: (i, 0)),
        ],
        out_shape=[
            jax.ShapeDtypeStruct((m, ze.shape[1]), jnp.float32),
            jax.ShapeDtypeStruct((m, out_d), jnp.float32),
        ],
        compiler_params=pltpu.CompilerParams(
            vmem_limit_bytes=110 * 1024 * 1024,
        ),
    )(ze, zq, W0, b0, W1, b1, W2, b2)


def _rowsq64(x2):
    p = x2[:, 0:8]
    for j in range(1, 8):
        p = p + x2[:, 8 * j:8 * j + 8]
    t = p[:, 0:4] + p[:, 4:8]
    t = t[:, 0:2] + t[:, 2:4]
    return t[:, 0:1] + t[:, 1:2]


def _colsq64(y2):
    p = y2[0:8, :]
    for j in range(1, 8):
        p = p + y2[8 * j:8 * j + 8, :]
    t = p[0:4, :] + p[4:8, :]
    t = t[0:2, :] + t[2:4, :]
    return t[0:1, :] + t[1:2, :]


def _vq_body(f_ref, cb_ref, cbT_ref, idx_ref):
    f = f_ref[...]
    cb = cb_ref[...]
    cbT = cbT_ref[...]
    dn = (((1,), (1,)), ((), ()))
    mm = lax.dot_general(f, cb, dn, preferred_element_type=jnp.float32)
    ss = _rowsq64(f * f)
    cc = _colsq64(cbT * cbT)
    d2 = ss + cc - 2.0 * mm
    dist = jnp.sqrt(jnp.maximum(d2, 0.0))
    m = jnp.min(dist, axis=1, keepdims=True)
    iota = lax.broadcasted_iota(jnp.int32, dist.shape, 1)
    hit = jnp.where(dist == m, iota, jnp.int32(dist.shape[1]))
    idx_ref[...] = jnp.min(hit, axis=1).astype(jnp.int32)


def _vq_call(flat, cb, cbT):
    n = flat.shape[0]
    grid = (n // VQBLK,)
    return pl.pallas_call(
        _vq_body,
        grid=grid,
        in_specs=[
            pl.BlockSpec((VQBLK, D), lambda i: (i, 0)),
            pl.BlockSpec((K, D), lambda i: (0, 0)),
            pl.BlockSpec((D, K), lambda i: (0, 0)),
        ],
        out_specs=pl.BlockSpec((VQBLK,), lambda i: (i,)),
        out_shape=jax.ShapeDtypeStruct((n,), jnp.int32),
        compiler_params=pltpu.CompilerParams(
            vmem_limit_bytes=110 * 1024 * 1024,
        ),
    )(flat, cb, cbT)


def _encvq_body(x_ref, W0_ref, b0_ref, W1_ref, b1_ref, W2_ref, b2_ref,
                cb_ref, cbT_ref, ze_ref, idx_ref):
    dn = (((1,), (1,)), ((), ()))
    h = lax.dot_general(x_ref[...], W0_ref[...], dn,
                        preferred_element_type=jnp.float32)
    h = jnp.maximum(h + b0_ref[...][None, :], 0.0)
    h = lax.dot_general(h, W1_ref[...], dn, preferred_element_type=jnp.float32)
    h = jnp.maximum(h + b1_ref[...][None, :], 0.0)
    h = lax.dot_general(h, W2_ref[...], dn, preferred_element_type=jnp.float32)
    z = h + b2_ref[...][None, :]
    ze_ref[...] = z
    cb = cb_ref[...]
    cbT = cbT_ref[...]
    cc = _colsq64(cbT * cbT)
    cols = []
    for j in range(NB_Z):
        f = z[:, j * D:(j + 1) * D]
        mm = lax.dot_general(f, cb, dn, preferred_element_type=jnp.float32)
        ss = _rowsq64(f * f)
        d2 = ss + cc - 2.0 * mm
        dist = jnp.sqrt(jnp.maximum(d2, 0.0))
        m = jnp.min(dist, axis=1, keepdims=True)
        iota = lax.broadcasted_iota(jnp.int32, dist.shape, 1)
        hit = jnp.where(dist == m, iota, jnp.int32(dist.shape[1]))
        cols.append(jnp.min(hit, axis=1, keepdims=True).astype(jnp.int32))
    idx_ref[...] = jnp.concatenate(cols, axis=1)


def _encvq_call(x, W0, b0, W1, b1, W2, b2, cb, cbT):
    m = x.shape[0]
    out_d = W2.shape[0]
    grid = (m // MBLK,)
    return pl.pallas_call(
        _encvq_body,
        grid=grid,
        in_specs=[
            pl.BlockSpec((MBLK, x.shape[1]), lambda i: (i, 0)),
            pl.BlockSpec(W0.shape, lambda i: (0, 0)),
            pl.BlockSpec(b0.shape, lambda i: (0,)),
            pl.BlockSpec(W1.shape, lambda i: (0, 0)),
            pl.BlockSpec(b1.shape, lambda i: (0,)),
            pl.BlockSpec(W2.shape, lambda i: (0, 0)),
            pl.BlockSpec(b2.shape, lambda i: (0,)),
            pl.BlockSpec((K, D), lambda i: (0, 0)),
            pl.BlockSpec((D, K), lambda i: (0, 0)),
        ],
        out_specs=[
            pl.BlockSpec((MBLK, out_d), lambda i: (i, 0)),
            pl.BlockSpec((MBLK, NB_Z), lambda i: (i, 0)),
        ],
        out_shape=[
            jax.ShapeDtypeStruct((m, out_d), jnp.float32),
            jax.ShapeDtypeStruct((m, NB_Z), jnp.int32),
        ],
        compiler_params=pltpu.CompilerParams(
            vmem_limit_bytes=110 * 1024 * 1024,
        ),
    )(x, W0, b0, W1, b1, W2, b2, cb, cbT)


def _sc_gather(cb, idx):
    info = plsc.get_sparse_core_info()
    nw = info.num_cores * info.num_subcores
    n = idx.shape[0]
    b_per_w = n // nw
    mesh = plsc.VectorSubcoreMesh(core_axis_name="c", subcore_axis_name="s")

    @functools.partial(
        pl.kernel,
        mesh=mesh,
        out_type=jax.ShapeDtypeStruct((n, D), jnp.float32),
        scratch_types=[
            pltpu.VMEM((b_per_w,), jnp.int32),
            pltpu.VMEM((b_per_w, D), jnp.float32),
            pltpu.SemaphoreType.DMA,
        ],
        compiler_params=pltpu.CompilerParams(use_tc_tiling_on_sc=False),
    )
    def gather_kernel(table_hbm, idx_hbm, out_hbm, idx_v, rows_v, sem):
        wid = lax.axis_index("s") * info.num_cores + lax.axis_index("c")
        base = wid * b_per_w
        pltpu.sync_copy(idx_hbm.at[pl.ds(base, b_per_w)], idx_v)
        pltpu.async_copy(table_hbm.at[idx_v], rows_v, sem).wait()
        pltpu.sync_copy(rows_v, out_hbm.at[pl.ds(base, b_per_w)])

    return gather_kernel(cb, idx)


def kernel(x, codebook, eW0, eb0, eW1, eb1, eW2, eb2,
           dW0, db0, dW1, db1, dW2, db2):
    ze_flat, indices2d = _encvq_call(x, eW0, eb0, eW1, eb1, eW2, eb2,
                                     codebook, codebook.T)
    indices = indices2d.reshape(-1)
    zq_flat = _sc_gather(codebook, indices)
    st_flat, x_hat = _dec_call(ze_flat, zq_flat.reshape(BATCH, NB_Z * D),
                               dW0, db0, dW1, db1, dW2, db2)
    shape3 = (BATCH, NB_Z, D)
    return (ze_flat.reshape(shape3),
            st_flat.reshape(shape3),
            zq_flat.reshape(shape3),
            x_hat,
            indices.reshape(BATCH, NB_Z))
